# baseline (speedup 1.0000x reference)
import functools

import jax
import jax.numpy as jnp
from jax.experimental import pallas as pl

N_TOKENS = 2048
INPUT_DIM = 2048
HIDDEN_DIM = 16384
TOPK = 64

ENC_TH = 512
MSK_TM = 128
DEC_TH = 512


def _encode_kernel(x_ref, we_ref, be_ref, out_ref):
    xb = x_ref[...].astype(jnp.bfloat16)
    wb = we_ref[...].astype(jnp.bfloat16)
    acc = jax.lax.dot_general(xb, wb, (((1,), (1,)), ((), ())),
                              preferred_element_type=jnp.float32)
    out_ref[...] = jnp.maximum(acc + be_ref[...], 0.0)


def _threshold_kernel(p_ref, t_ref):
    v = p_ref[...]
    bits = jax.lax.bitcast_convert_type(v, jnp.int32)
    hi = jnp.max(bits, axis=1, keepdims=True)
    lo = jnp.zeros_like(hi)

    def body(_, carry):
        lo, hi = carry
        mid = lo + ((hi - lo + 1) >> 1)
        cnt = jnp.sum((bits >= mid).astype(jnp.int32), axis=1, keepdims=True)
        ge = cnt >= TOPK
        return jnp.where(ge, mid, lo), jnp.where(ge, hi, mid - 1)

    lo, hi = jax.lax.fori_loop(0, 31, body, (lo, hi))
    t_ref[...] = jax.lax.bitcast_convert_type(lo, jnp.float32)


def _mask_decode_kernel(p_ref, t_ref, wd_ref, bd_ref, out_ref, z_ref):
    h = pl.program_id(0)
    pre = p_ref[...]
    zb = jnp.where(pre >= t_ref[...], pre, 0.0)
    z_ref[...] = zb
    part = jax.lax.dot_general(zb.astype(jnp.bfloat16),
                               wd_ref[...].astype(jnp.bfloat16),
                               (((1,), (1,)), ((), ())),
                               preferred_element_type=jnp.float32)

    @pl.when(h == 0)
    def _():
        out_ref[...] = part + bd_ref[...]

    @pl.when(h > 0)
    def _():
        out_ref[...] += part


@functools.partial(jax.jit, static_argnames=("interpret",))
def kernel(x, We, be, Wd, bd, interpret=False):
    pre_act = pl.pallas_call(
        _encode_kernel,
        grid=(HIDDEN_DIM // ENC_TH,),
        in_specs=[
            pl.BlockSpec((N_TOKENS, INPUT_DIM), lambda h: (0, 0)),
            pl.BlockSpec((ENC_TH, INPUT_DIM), lambda h: (h, 0)),
            pl.BlockSpec((ENC_TH,), lambda h: (h,)),
        ],
        out_specs=pl.BlockSpec((N_TOKENS, ENC_TH), lambda h: (0, h)),
        out_shape=jax.ShapeDtypeStruct((N_TOKENS, HIDDEN_DIM), jnp.float32),
        interpret=interpret,
    )(x, We, be)

    thr = pl.pallas_call(
        _threshold_kernel,
        grid=(N_TOKENS // MSK_TM,),
        in_specs=[pl.BlockSpec((MSK_TM, HIDDEN_DIM), lambda r: (r, 0))],
        out_specs=pl.BlockSpec((MSK_TM, 1), lambda r: (r, 0)),
        out_shape=jax.ShapeDtypeStruct((N_TOKENS, 1), jnp.float32),
        interpret=interpret,
    )(pre_act)

    x_hat, z = pl.pallas_call(
        _mask_decode_kernel,
        grid=(HIDDEN_DIM // DEC_TH,),
        in_specs=[
            pl.BlockSpec((N_TOKENS, DEC_TH), lambda h: (0, h)),
            pl.BlockSpec((N_TOKENS, 1), lambda h: (0, 0)),
            pl.BlockSpec((INPUT_DIM, DEC_TH), lambda h: (0, h)),
            pl.BlockSpec((INPUT_DIM,), lambda h: (0,)),
        ],
        out_specs=[
            pl.BlockSpec((N_TOKENS, INPUT_DIM), lambda h: (0, 0)),
            pl.BlockSpec((N_TOKENS, DEC_TH), lambda h: (0, h)),
        ],
        out_shape=[
            jax.ShapeDtypeStruct((N_TOKENS, INPUT_DIM), jnp.float32),
            jax.ShapeDtypeStruct((N_TOKENS, HIDDEN_DIM), jnp.float32),
        ],
        interpret=interpret,
    )(pre_act, thr, Wd, bd)

    return (x_hat, z)
